# baseline (speedup 1.0000x reference)
import jax
import jax.numpy as jnp
from jax import lax
from jax.experimental import pallas as pl
from jax.experimental.pallas import tpu as pltpu
from jax.experimental.pallas import tpu_sc as plsc

SEQ_LEN = 200
BATCH = 4096
VOCAB = 1000000
EMBED_DIM = 64
OUTPUT_DIM = 2
NUM_CORES = 2
NUM_SUBCORES = 16
NUM_WORKERS = NUM_CORES * NUM_SUBCORES
B_PER_W = BATCH // NUM_WORKERS
PROJ_BLK = 16384


def _proj_body(w_ref, x_ref, b_ref, o0_ref, o1_ref):
  inv = jnp.float32(1.0 / SEQ_LEN)
  y = jnp.dot(w_ref[...] * inv, x_ref[...],
              preferred_element_type=jnp.float32)
  o0_ref[...] = y[0] + b_ref[0] * inv
  o1_ref[...] = y[1] + b_ref[1] * inv


def _project_table(table_t, fc_w, fc_b):
  grid = (VOCAB + PROJ_BLK - 1) // PROJ_BLK
  vec = jax.ShapeDtypeStruct((VOCAB,), jnp.float32)
  return pl.pallas_call(
      _proj_body,
      grid=(grid,),
      in_specs=[
          pl.BlockSpec((OUTPUT_DIM, EMBED_DIM), lambda i: (0, 0)),
          pl.BlockSpec((EMBED_DIM, PROJ_BLK), lambda i: (0, i)),
          pl.BlockSpec(memory_space=pltpu.SMEM),
      ],
      out_specs=[
          pl.BlockSpec((PROJ_BLK,), lambda i: (i,)),
          pl.BlockSpec((PROJ_BLK,), lambda i: (i,)),
      ],
      out_shape=[vec, vec],
  )(fc_w, table_t, fc_b)


LANES = 16
N_ACC = OUTPUT_DIM * B_PER_W // LANES


NBUF = 5


def _sc_body(text_ref, p0_ref, p1_ref, out_ref, idx_v, *bufs_sems):
  bufs = bufs_sems[:NBUF]
  sems = bufs_sems[NBUF:]
  sid = lax.axis_index("s")
  wid = sid * NUM_CORES + lax.axis_index("c")
  base = wid * B_PER_W

  pltpu.sync_copy(text_ref.at[:, pl.ds(base, B_PER_W)], idx_v)

  def start(t, j):
    pltpu.async_copy(p0_ref.at[idx_v.at[t]], bufs[j].at[0], sems[j])
    pltpu.async_copy(p1_ref.at[idx_v.at[t]], bufs[j].at[1], sems[j])

  def wait(j):
    pltpu.make_async_copy(p0_ref.at[idx_v.at[0]], bufs[j].at[0], sems[j]).wait()
    pltpu.make_async_copy(p1_ref.at[idx_v.at[0]], bufs[j].at[1], sems[j]).wait()

  def loads(j):
    flat = []
    for o in range(OUTPUT_DIM):
      for c in range(B_PER_W // LANES):
        flat.append(bufs[j][o, pl.ds(c * LANES, LANES)])
    return flat

  def add(acc, j):
    return [a + v for a, v in zip(acc, loads(j))]

  for j in range(NBUF):
    start(j, j)
  wait(0)
  acc = loads(0)
  start(NBUF, 0)
  for j in range(1, NBUF):
    wait(j)
    acc = add(acc, j)
    start(NBUF + j, j)

  def body(g, acc):
    for j in range(NBUF):
      wait(j)
      acc = add(acc, j)
      start(NBUF * g + NBUF + j, j)
    return acc

  acc = lax.fori_loop(1, SEQ_LEN // NBUF - 1, body, acc)

  for j in range(NBUF):
    wait(j)
    acc = add(acc, j)

  k = 0
  for o in range(OUTPUT_DIM):
    for c in range(B_PER_W // LANES):
      bufs[0][o, pl.ds(c * LANES, LANES)] = acc[k]
      k += 1
  pltpu.sync_copy(bufs[0], out_ref.at[:, pl.ds(base, B_PER_W)])


def _sc_embed_bag(text, proj0, proj1):
  mesh = plsc.VectorSubcoreMesh(core_axis_name="c", subcore_axis_name="s")
  return pl.kernel(
      _sc_body,
      out_type=jax.ShapeDtypeStruct((OUTPUT_DIM, BATCH), jnp.float32),
      mesh=mesh,
      scratch_types=[
          pltpu.VMEM((SEQ_LEN, B_PER_W), jnp.int32),
      ] + [
          pltpu.VMEM((OUTPUT_DIM, B_PER_W), jnp.float32)
          for _ in range(NBUF)
      ] + [
          pltpu.SemaphoreType.DMA
          for _ in range(NBUF)
      ],
      compiler_params=pltpu.CompilerParams(use_tc_tiling_on_sc=False),
  )(text, proj0, proj1)


@jax.jit
def kernel(text, embed_table, fc_w, fc_b):
  text = text.astype(jnp.int32)
  proj0, proj1 = _project_table(embed_table.T, fc_w, fc_b)
  out = _sc_embed_bag(text, proj0, proj1)
  return out.T


# speedup vs baseline: 1.0603x; 1.0603x over previous
import jax
import jax.numpy as jnp
from jax import lax
from jax.experimental import pallas as pl
from jax.experimental.pallas import tpu as pltpu
from jax.experimental.pallas import tpu_sc as plsc

SEQ_LEN = 200
BATCH = 4096
VOCAB = 1000000
EMBED_DIM = 64
OUTPUT_DIM = 2
NUM_CORES = 2
NUM_SUBCORES = 16
NUM_WORKERS = NUM_CORES * NUM_SUBCORES
B_PER_W = BATCH // NUM_WORKERS
PROJ_BLK = 49152


def _proj_body(w_ref, x_ref, b_ref, o0_ref, o1_ref):
  inv = jnp.float32(1.0 / SEQ_LEN)
  y = jnp.dot(w_ref[...] * inv, x_ref[...],
              preferred_element_type=jnp.float32)
  o0_ref[...] = y[0] + b_ref[0] * inv
  o1_ref[...] = y[1] + b_ref[1] * inv


def _project_table(table_t, fc_w, fc_b):
  grid = (VOCAB + PROJ_BLK - 1) // PROJ_BLK
  vec = jax.ShapeDtypeStruct((VOCAB,), jnp.float32)
  return pl.pallas_call(
      _proj_body,
      grid=(grid,),
      in_specs=[
          pl.BlockSpec((OUTPUT_DIM, EMBED_DIM), lambda i: (0, 0)),
          pl.BlockSpec((EMBED_DIM, PROJ_BLK), lambda i: (0, i)),
          pl.BlockSpec(memory_space=pltpu.SMEM),
      ],
      out_specs=[
          pl.BlockSpec((PROJ_BLK,), lambda i: (i,)),
          pl.BlockSpec((PROJ_BLK,), lambda i: (i,)),
      ],
      out_shape=[vec, vec],
  )(fc_w, table_t, fc_b)


LANES = 16
N_ACC = OUTPUT_DIM * B_PER_W // LANES


NBUF = 5


def _sc_body(text_ref, p0_ref, p1_ref, out_ref, idx_v, *bufs_sems):
  bufs = bufs_sems[:NBUF]
  sems = bufs_sems[NBUF:]
  sid = lax.axis_index("s")
  wid = sid * NUM_CORES + lax.axis_index("c")
  base = wid * B_PER_W

  pltpu.sync_copy(text_ref.at[:, pl.ds(base, B_PER_W)], idx_v)

  def start(t, j):
    pltpu.async_copy(p0_ref.at[idx_v.at[t]], bufs[j].at[0], sems[j])
    pltpu.async_copy(p1_ref.at[idx_v.at[t]], bufs[j].at[1], sems[j])

  def wait(j):
    pltpu.make_async_copy(p0_ref.at[idx_v.at[0]], bufs[j].at[0], sems[j]).wait()
    pltpu.make_async_copy(p1_ref.at[idx_v.at[0]], bufs[j].at[1], sems[j]).wait()

  def loads(j):
    flat = []
    for o in range(OUTPUT_DIM):
      for c in range(B_PER_W // LANES):
        flat.append(bufs[j][o, pl.ds(c * LANES, LANES)])
    return flat

  def add(acc, j):
    return [a + v for a, v in zip(acc, loads(j))]

  for j in range(NBUF):
    start(j, j)
  wait(0)
  acc = loads(0)
  start(NBUF, 0)
  for j in range(1, NBUF):
    wait(j)
    acc = add(acc, j)
    start(NBUF + j, j)

  def body(g, acc):
    for j in range(NBUF):
      wait(j)
      acc = add(acc, j)
      start(NBUF * g + NBUF + j, j)
    return acc

  acc = lax.fori_loop(1, SEQ_LEN // NBUF - 1, body, acc)

  for j in range(NBUF):
    wait(j)
    acc = add(acc, j)

  k = 0
  for o in range(OUTPUT_DIM):
    for c in range(B_PER_W // LANES):
      bufs[0][o, pl.ds(c * LANES, LANES)] = acc[k]
      k += 1
  pltpu.sync_copy(bufs[0], out_ref.at[:, pl.ds(base, B_PER_W)])


def _sc_embed_bag(text, proj0, proj1):
  mesh = plsc.VectorSubcoreMesh(core_axis_name="c", subcore_axis_name="s")
  return pl.kernel(
      _sc_body,
      out_type=jax.ShapeDtypeStruct((OUTPUT_DIM, BATCH), jnp.float32),
      mesh=mesh,
      scratch_types=[
          pltpu.VMEM((SEQ_LEN, B_PER_W), jnp.int32),
      ] + [
          pltpu.VMEM((OUTPUT_DIM, B_PER_W), jnp.float32)
          for _ in range(NBUF)
      ] + [
          pltpu.SemaphoreType.DMA
          for _ in range(NBUF)
      ],
      compiler_params=pltpu.CompilerParams(use_tc_tiling_on_sc=False),
  )(text, proj0, proj1)


@jax.jit
def kernel(text, embed_table, fc_w, fc_b):
  text = text.astype(jnp.int32)
  proj0, proj1 = _project_table(embed_table.T, fc_w, fc_b)
  out = _sc_embed_bag(text, proj0, proj1)
  return out.T


# speedup vs baseline: 1.0637x; 1.0032x over previous
import jax
import jax.numpy as jnp
from jax import lax
from jax.experimental import pallas as pl
from jax.experimental.pallas import tpu as pltpu
from jax.experimental.pallas import tpu_sc as plsc

SEQ_LEN = 200
BATCH = 4096
VOCAB = 1000000
EMBED_DIM = 64
OUTPUT_DIM = 2
NUM_CORES = 2
NUM_SUBCORES = 16
NUM_WORKERS = NUM_CORES * NUM_SUBCORES
B_PER_W = BATCH // NUM_WORKERS
PROJ_BLK = 32768


def _proj_body(w_ref, x_ref, b_ref, o0_ref, o1_ref):
  inv = jnp.float32(1.0 / SEQ_LEN)
  y = jnp.dot(w_ref[...] * inv, x_ref[...],
              preferred_element_type=jnp.float32)
  o0_ref[...] = y[0] + b_ref[0] * inv
  o1_ref[...] = y[1] + b_ref[1] * inv


def _project_table(table_t, fc_w, fc_b):
  grid = (VOCAB + PROJ_BLK - 1) // PROJ_BLK
  vec = jax.ShapeDtypeStruct((VOCAB,), jnp.float32)
  return pl.pallas_call(
      _proj_body,
      grid=(grid,),
      in_specs=[
          pl.BlockSpec((OUTPUT_DIM, EMBED_DIM), lambda i: (0, 0)),
          pl.BlockSpec((EMBED_DIM, PROJ_BLK), lambda i: (0, i)),
          pl.BlockSpec(memory_space=pltpu.SMEM),
      ],
      out_specs=[
          pl.BlockSpec((PROJ_BLK,), lambda i: (i,)),
          pl.BlockSpec((PROJ_BLK,), lambda i: (i,)),
      ],
      out_shape=[vec, vec],
  )(fc_w, table_t, fc_b)


LANES = 16
N_ACC = OUTPUT_DIM * B_PER_W // LANES


NBUF = 5


def _sc_body(text_ref, p0_ref, p1_ref, out_ref, idx_v, *bufs_sems):
  bufs = bufs_sems[:NBUF]
  sems = bufs_sems[NBUF:]
  sid = lax.axis_index("s")
  wid = sid * NUM_CORES + lax.axis_index("c")
  base = wid * B_PER_W

  pltpu.sync_copy(text_ref.at[:, pl.ds(base, B_PER_W)], idx_v)

  def start(t, j):
    pltpu.async_copy(p0_ref.at[idx_v.at[t]], bufs[j].at[0], sems[j])
    pltpu.async_copy(p1_ref.at[idx_v.at[t]], bufs[j].at[1], sems[j])

  def wait(j):
    pltpu.make_async_copy(p0_ref.at[idx_v.at[0]], bufs[j].at[0], sems[j]).wait()
    pltpu.make_async_copy(p1_ref.at[idx_v.at[0]], bufs[j].at[1], sems[j]).wait()

  def loads(j):
    flat = []
    for o in range(OUTPUT_DIM):
      for c in range(B_PER_W // LANES):
        flat.append(bufs[j][o, pl.ds(c * LANES, LANES)])
    return flat

  def add(acc, j):
    return [a + v for a, v in zip(acc, loads(j))]

  for j in range(NBUF):
    start(j, j)
  wait(0)
  acc = loads(0)
  start(NBUF, 0)
  for j in range(1, NBUF):
    wait(j)
    acc = add(acc, j)
    start(NBUF + j, j)

  def body(g, acc):
    for j in range(NBUF):
      wait(j)
      acc = add(acc, j)
      start(NBUF * g + NBUF + j, j)
    return acc

  acc = lax.fori_loop(1, SEQ_LEN // NBUF - 1, body, acc)

  for j in range(NBUF):
    wait(j)
    acc = add(acc, j)

  k = 0
  for o in range(OUTPUT_DIM):
    for c in range(B_PER_W // LANES):
      bufs[0][o, pl.ds(c * LANES, LANES)] = acc[k]
      k += 1
  pltpu.sync_copy(bufs[0], out_ref.at[:, pl.ds(base, B_PER_W)])


def _sc_embed_bag(text, proj0, proj1):
  mesh = plsc.VectorSubcoreMesh(core_axis_name="c", subcore_axis_name="s")
  return pl.kernel(
      _sc_body,
      out_type=jax.ShapeDtypeStruct((OUTPUT_DIM, BATCH), jnp.float32),
      mesh=mesh,
      scratch_types=[
          pltpu.VMEM((SEQ_LEN, B_PER_W), jnp.int32),
      ] + [
          pltpu.VMEM((OUTPUT_DIM, B_PER_W), jnp.float32)
          for _ in range(NBUF)
      ] + [
          pltpu.SemaphoreType.DMA
          for _ in range(NBUF)
      ],
      compiler_params=pltpu.CompilerParams(use_tc_tiling_on_sc=False),
  )(text, proj0, proj1)


@jax.jit
def kernel(text, embed_table, fc_w, fc_b):
  text = text.astype(jnp.int32)
  proj0, proj1 = _project_table(embed_table.T, fc_w, fc_b)
  out = _sc_embed_bag(text, proj0, proj1)
  return out.T
